# Initial kernel scaffold; baseline (speedup 1.0000x reference)
import jax
import jax.numpy as jnp
from jax.experimental import pallas as pl


def kernel(x):
    raise NotImplementedError("write your pallas kernel here")


# speedup vs baseline: 149.5901x; 149.5901x over previous
import functools

import jax
import jax.numpy as jnp
from jax import lax
from jax.experimental import pallas as pl
from jax.experimental.pallas import tpu as pltpu
from jax.experimental.pallas import tpu_sc as plsc

ROWS, COLS = 8192, 4096
TOTAL = ROWS * COLS
BLOCK = 8
KEEP = 4

NC, NS, L = 2, 16, 16
NW = NC * NS
PER_W = TOTAL // NW
CHUNK = 16384
NCHUNK = PER_W // CHUNK
NBUF = 4
WIN = L * BLOCK
NWIN = CHUNK // WIN


def _chunk_slice(ref, base, g):
    return ref.at[pl.ds(base + g * CHUNK, CHUNK)]


def _compute_chunk(buf, iota8):
    def body(t, carry):
        cb = t * WIN
        idx = [iota8 + (cb + j) for j in range(BLOCK)]
        v = [plsc.load_gather(buf, [idx[j]]) for j in range(BLOCK)]
        a = [jnp.abs(vj) for vj in v]
        pos = [jnp.full((L,), BLOCK - 1 - j, jnp.int32) for j in range(BLOCK)]
        for j in range(BLOCK):
            for i in range(j):
                c = (a[j] >= a[i]).astype(jnp.int32)
                pos[j] = pos[j] + c
                pos[i] = pos[i] - c
        for j in range(BLOCK):
            out = jnp.where(pos[j] >= KEEP, v[j], 0.0)
            plsc.store_scatter(buf, [idx[j]], out)
        return carry

    lax.fori_loop(0, NWIN, body, None)


@functools.partial(
    pl.kernel,
    out_type=jax.ShapeDtypeStruct((TOTAL,), jnp.float32),
    mesh=plsc.VectorSubcoreMesh(core_axis_name="c", subcore_axis_name="s"),
    compiler_params=pltpu.CompilerParams(needs_layout_passes=False),
    scratch_types=(
        [pltpu.VMEM((CHUNK,), jnp.float32) for _ in range(NBUF)]
        + [pltpu.SemaphoreType.DMA for _ in range(2 * NBUF)]
    ),
)
def _sparsify_sc(x_hbm, o_hbm, b0, b1, b2, b3,
                 si0, si1, si2, si3, so0, so1, so2, so3):
    bufs = [b0, b1, b2, b3]
    si = [si0, si1, si2, si3]
    so = [so0, so1, so2, so3]

    wid = lax.axis_index("s") * NC + lax.axis_index("c")
    base = wid * PER_W
    iota8 = lax.iota(jnp.int32, L) * BLOCK

    def start_in(g, b):
        pltpu.async_copy(_chunk_slice(x_hbm, base, g), bufs[b], si[b])

    def wait_in(b):
        pltpu.make_async_copy(_chunk_slice(x_hbm, base, 0), bufs[b], si[b]).wait()

    def start_out(g, b):
        pltpu.async_copy(bufs[b], _chunk_slice(o_hbm, base, g), so[b])

    def wait_out(b):
        pltpu.make_async_copy(bufs[b], _chunk_slice(o_hbm, base, 0), so[b]).wait()

    start_in(0, 0)
    start_in(1, 1)

    def outer(ho, carry):
        for b in range(NBUF):
            h = ho * NBUF + b
            wait_in(b)
            _compute_chunk(bufs[b], iota8)
            start_out(h, b)
            nxt = h + 2
            bn = (b + 2) % NBUF

            @pl.when(nxt < NCHUNK)
            def _():
                @pl.when(nxt >= NBUF)
                def _():
                    wait_out(bn)

                start_in(nxt, bn)

        return carry

    lax.fori_loop(0, NCHUNK // NBUF, outer, None)

    for b in range(NBUF):
        wait_out(b)


def kernel(x):
    return _sparsify_sc(x.reshape(TOTAL)).reshape(ROWS, COLS)


# speedup vs baseline: 232.8759x; 1.5568x over previous
import functools

import jax
import jax.numpy as jnp
from jax import lax
from jax.experimental import pallas as pl
from jax.experimental.pallas import tpu as pltpu
from jax.experimental.pallas import tpu_sc as plsc

ROWS, COLS = 8192, 4096
BLOCK = 8
KEEP = 4

NC, NS, L = 2, 16, 16
NW = NC * NS
ROWS_PER_W = ROWS // NW
CHUNK_ROWS = 4
NCHUNK = ROWS_PER_W // CHUNK_ROWS
NBUF = 4
WIN = L * BLOCK
WPR = COLS // WIN
NWIN = CHUNK_ROWS * WPR


def _chunk_slice(ref, row0, g):
    return ref.at[pl.ds(row0 + g * CHUNK_ROWS, CHUNK_ROWS), :]


def _compute_chunk(buf, iota8):
    def body(t, carry):
        r = jnp.full((L,), t // WPR, jnp.int32)
        cb = (t % WPR) * WIN
        idx = [iota8 + (cb + j) for j in range(BLOCK)]
        v = [plsc.load_gather(buf, [r, idx[j]]) for j in range(BLOCK)]
        a = [jnp.abs(vj) for vj in v]
        pos = [jnp.full((L,), BLOCK - 1 - j, jnp.int32) for j in range(BLOCK)]
        for j in range(BLOCK):
            for i in range(j):
                c = (a[j] >= a[i]).astype(jnp.int32)
                pos[j] = pos[j] + c
                pos[i] = pos[i] - c
        for j in range(BLOCK):
            out = jnp.where(pos[j] >= KEEP, v[j], 0.0)
            plsc.store_scatter(buf, [r, idx[j]], out)
        return carry

    lax.fori_loop(0, NWIN, body, None)


@functools.partial(
    pl.kernel,
    out_type=jax.ShapeDtypeStruct((ROWS, COLS), jnp.float32),
    mesh=plsc.VectorSubcoreMesh(core_axis_name="c", subcore_axis_name="s"),
    compiler_params=pltpu.CompilerParams(needs_layout_passes=False),
    scratch_types=(
        [pltpu.VMEM((CHUNK_ROWS, COLS), jnp.float32) for _ in range(NBUF)]
        + [pltpu.SemaphoreType.DMA for _ in range(2 * NBUF)]
    ),
)
def _sparsify_sc(x_hbm, o_hbm, b0, b1, b2, b3,
                 si0, si1, si2, si3, so0, so1, so2, so3):
    bufs = [b0, b1, b2, b3]
    si = [si0, si1, si2, si3]
    so = [so0, so1, so2, so3]

    wid = lax.axis_index("s") * NC + lax.axis_index("c")
    row0 = wid * ROWS_PER_W
    iota8 = lax.iota(jnp.int32, L) * BLOCK

    def start_in(g, b):
        pltpu.async_copy(_chunk_slice(x_hbm, row0, g), bufs[b], si[b])

    def wait_in(b):
        pltpu.make_async_copy(_chunk_slice(x_hbm, row0, 0), bufs[b], si[b]).wait()

    def start_out(g, b):
        pltpu.async_copy(bufs[b], _chunk_slice(o_hbm, row0, g), so[b])

    def wait_out(b):
        pltpu.make_async_copy(bufs[b], _chunk_slice(o_hbm, row0, 0), so[b]).wait()

    start_in(0, 0)
    start_in(1, 1)

    def outer(ho, carry):
        for b in range(NBUF):
            h = ho * NBUF + b
            wait_in(b)
            _compute_chunk(bufs[b], iota8)
            start_out(h, b)
            nxt = h + 2
            bn = (b + 2) % NBUF

            @pl.when(nxt < NCHUNK)
            def _():
                @pl.when(nxt >= NBUF)
                def _():
                    wait_out(bn)

                start_in(nxt, bn)

        return carry

    lax.fori_loop(0, NCHUNK // NBUF, outer, None)

    for b in range(NBUF):
        wait_out(b)


def kernel(x):
    return _sparsify_sc(x)
